# Initial kernel scaffold; baseline (speedup 1.0000x reference)
import jax
import jax.numpy as jnp
from jax.experimental import pallas as pl


def kernel(h, A, r_count, w, a_src_dst):
    raise NotImplementedError("write your pallas kernel here")


# speedup vs baseline: 2.1566x; 2.1566x over previous
import functools

import jax
import jax.numpy as jnp
from jax import lax
from jax.experimental import pallas as pl
from jax.experimental.pallas import tpu as pltpu
from jax.experimental.pallas import tpu_sc as plsc

N = 10000
F = 128
E = 160000
NPAD = 10112
EPAD = 163840
C = 128
NCHUNK = EPAD // C
CH_PER_TILE = NCHUNK // 32
ROWS_PER_TILE = NPAD // 16
NB = 4
BR = NPAD // NB

_mesh = functools.partial(
    plsc.VectorSubcoreMesh, core_axis_name="c", subcore_axis_name="s",
    num_cores=2, num_subcores=16)


def _sc_pass1(h_pad, src2d, dst2d, rel2d, zeros_rows):
    @functools.partial(
        pl.kernel,
        out_type=jax.ShapeDtypeStruct((2, NPAD, F), jnp.float32),
        mesh=_mesh(),
        scratch_types=[
            pltpu.VMEM((CH_PER_TILE, C), jnp.int32),
            pltpu.VMEM((CH_PER_TILE, C), jnp.int32),
            pltpu.VMEM((CH_PER_TILE, C), jnp.int32),
            pltpu.VMEM((C, F), jnp.float32),
            pltpu.VMEM((C, F), jnp.float32),
            pltpu.VMEM_SHARED((NPAD, F), jnp.float32),
            pltpu.SemaphoreType.DMA,
            pltpu.SemaphoreType.DMA,
        ],
    )
    def k(h_hbm, src_hbm, dst_hbm, rel_hbm, z_hbm, out_hbm,
          src_v, dst_v, rel_v, hs_v, hd_v, acc_sh, sem1, sem2):
        cid = lax.axis_index("c")
        sid = lax.axis_index("s")
        wid = sid * 2 + cid
        row0 = sid * ROWS_PER_TILE
        pltpu.sync_copy(src_hbm.at[pl.ds(wid * CH_PER_TILE, CH_PER_TILE), :], src_v)
        pltpu.sync_copy(dst_hbm.at[pl.ds(wid * CH_PER_TILE, CH_PER_TILE), :], dst_v)
        pltpu.sync_copy(rel_hbm.at[pl.ds(wid * CH_PER_TILE, CH_PER_TILE), :], rel_v)
        pltpu.sync_copy(z_hbm.at[pl.ds(row0, ROWS_PER_TILE), :],
                        acc_sh.at[pl.ds(row0, ROWS_PER_TILE), :])
        plsc.subcore_barrier()

        def chunk(j, carry):
            g1 = pltpu.async_copy(h_hbm.at[src_v.at[j]], hs_v, sem1)
            g2 = pltpu.async_copy(h_hbm.at[dst_v.at[j]], hd_v, sem2)
            g1.wait()
            g2.wait()

            def edge(e, c2):
                for kk in range(F // 16):
                    sl = pl.ds(kk * 16, 16)
                    hs_v[e, sl] = hs_v[e, sl] - hd_v[e, sl]
                return c2
            lax.fori_loop(0, C, edge, 0, unroll=2)
            pltpu.sync_copy(hs_v, acc_sh.at[rel_v.at[j]], add=True)
            return carry
        lax.fori_loop(0, CH_PER_TILE, chunk, 0)

        plsc.subcore_barrier()
        pltpu.sync_copy(acc_sh.at[pl.ds(row0, ROWS_PER_TILE), :],
                        out_hbm.at[cid, pl.ds(row0, ROWS_PER_TILE), :])

    return k(h_pad, src2d, dst2d, rel2d, zeros_rows)


def _tc_a(part, rc_pad):
    def body(part_ref, rc_ref, ir_ref, er_ref, c_ref):
        raw = part_ref[0] + part_ref[1]
        inv = 1.0 / rc_ref[...]
        ir = raw * inv
        er = ir * inv
        ir_ref[...] = ir
        er_ref[...] = er

        @pl.when(pl.program_id(0) == 0)
        def _():
            c_ref[...] = jnp.zeros_like(c_ref)
        c_ref[0:1, :] += jnp.sum(er, axis=0, keepdims=True)

    return pl.pallas_call(
        body,
        grid=(NB,),
        in_specs=[
            pl.BlockSpec((2, BR, F), lambda i: (0, i, 0)),
            pl.BlockSpec((BR, 1), lambda i: (i, 0)),
        ],
        out_specs=[
            pl.BlockSpec((BR, F), lambda i: (i, 0)),
            pl.BlockSpec((BR, F), lambda i: (i, 0)),
            pl.BlockSpec((8, F), lambda i: (0, 0)),
        ],
        out_shape=[
            jax.ShapeDtypeStruct((NPAD, F), jnp.float32),
            jax.ShapeDtypeStruct((NPAD, F), jnp.float32),
            jax.ShapeDtypeStruct((8, F), jnp.float32),
        ],
    )(part, rc_pad)


def _tc_b(inputr, edge_r, csum):
    def body(ir_ref, er_ref, c_ref, g_ref):
        er = er_ref[...]
        cvec = c_ref[0:1, :]
        rowsum = jax.lax.dot_general(
            er, cvec, (((1,), (1,)), ((), ())),
            preferred_element_type=jnp.float32, precision=jax.lax.Precision.HIGHEST)
        u = jnp.where(rowsum != 0.0, ir_ref[...] / rowsum, 0.0)

        @pl.when(pl.program_id(0) == 0)
        def _():
            g_ref[...] = jnp.zeros_like(g_ref)
        g_ref[...] += jax.lax.dot_general(
            er, u, (((0,), (0,)), ((), ())),
            preferred_element_type=jnp.float32, precision=jax.lax.Precision.HIGHEST)

    return pl.pallas_call(
        body,
        grid=(NB,),
        in_specs=[
            pl.BlockSpec((BR, F), lambda i: (i, 0)),
            pl.BlockSpec((BR, F), lambda i: (i, 0)),
            pl.BlockSpec((8, F), lambda i: (0, 0)),
        ],
        out_specs=pl.BlockSpec((F, F), lambda i: (0, 0)),
        out_shape=jax.ShapeDtypeStruct((F, F), jnp.float32),
    )(inputr, edge_r, csum)


def _tc_c(edge_r, g, h_pad, a_flat):
    def body(er_ref, g_ref, h_ref, af_ref, ir2_ref, sc_ref):
        er = er_ref[...]
        ir2 = jnp.dot(er, g_ref[...], preferred_element_type=jnp.float32, precision=jax.lax.Precision.HIGHEST)
        ir2_ref[...] = ir2
        hblk = h_ref[...]
        af = af_ref[...]
        w0 = af[4:5, :]
        p0 = jax.lax.dot_general(hblk, af[0:1, :], (((1,), (1,)), ((), ())),
                                 preferred_element_type=jnp.float32, precision=jax.lax.Precision.HIGHEST)
        p1 = jax.lax.dot_general(hblk * w0, af[1:2, :], (((1,), (1,)), ((), ())),
                                 preferred_element_type=jnp.float32, precision=jax.lax.Precision.HIGHEST)
        q0 = jax.lax.dot_general(ir2, af[2:3, :], (((1,), (1,)), ((), ())),
                                 preferred_element_type=jnp.float32, precision=jax.lax.Precision.HIGHEST)
        q1 = jax.lax.dot_general(ir2, af[3:4, :], (((1,), (1,)), ((), ())),
                                 preferred_element_type=jnp.float32, precision=jax.lax.Precision.HIGHEST)
        z = jnp.zeros_like(p0)
        sc_ref[...] = jnp.concatenate([p0, p1, q0, q1, z, z, z, z], axis=1)

    return pl.pallas_call(
        body,
        grid=(NB,),
        in_specs=[
            pl.BlockSpec((BR, F), lambda i: (i, 0)),
            pl.BlockSpec((F, F), lambda i: (0, 0)),
            pl.BlockSpec((BR, F), lambda i: (i, 0)),
            pl.BlockSpec((8, F), lambda i: (0, 0)),
        ],
        out_specs=[
            pl.BlockSpec((BR, F), lambda i: (i, 0)),
            pl.BlockSpec((BR, 8), lambda i: (i, 0)),
        ],
        out_shape=[
            jax.ShapeDtypeStruct((NPAD, F), jnp.float32),
            jax.ShapeDtypeStruct((NPAD, 8), jnp.float32),
        ],
    )(edge_r, g, h_pad, a_flat)


def _sc_scores(scores_flat, src2d, dst2d, rel2d):
    @functools.partial(
        pl.kernel,
        out_type=(
            jax.ShapeDtypeStruct((2 * EPAD,), jnp.float32),
            jax.ShapeDtypeStruct((2 * 32 * NPAD,), jnp.float32),
        ),
        mesh=_mesh(),
        scratch_types=[
            pltpu.VMEM((CH_PER_TILE, C), jnp.int32),
            pltpu.VMEM((CH_PER_TILE, C), jnp.int32),
            pltpu.VMEM((CH_PER_TILE, C), jnp.int32),
            pltpu.VMEM((NPAD,), jnp.float32),
            pltpu.VMEM((NPAD,), jnp.float32),
            pltpu.VMEM((NPAD,), jnp.float32),
            pltpu.VMEM((NPAD,), jnp.float32),
            pltpu.VMEM((NPAD,), jnp.float32),
            pltpu.VMEM((NPAD,), jnp.float32),
            pltpu.VMEM((C,), jnp.float32),
            pltpu.VMEM((C,), jnp.float32),
        ],
        compiler_params=pltpu.CompilerParams(needs_layout_passes=False),
    )
    def k(sc_hbm, src_hbm, dst_hbm, rel_hbm, ee_out, rs_out,
          src_v, dst_v, rel_v, p0_t, q0_t, p1_t, q1_t, rs0, rs1, ee0_v, ee1_v):
        cid = lax.axis_index("c")
        sid = lax.axis_index("s")
        wid = sid * 2 + cid
        pltpu.sync_copy(src_hbm.at[pl.ds(wid * CH_PER_TILE, CH_PER_TILE), :], src_v)
        pltpu.sync_copy(dst_hbm.at[pl.ds(wid * CH_PER_TILE, CH_PER_TILE), :], dst_v)
        pltpu.sync_copy(rel_hbm.at[pl.ds(wid * CH_PER_TILE, CH_PER_TILE), :], rel_v)
        pltpu.sync_copy(sc_hbm.at[pl.ds(0 * NPAD, NPAD)], p0_t)
        pltpu.sync_copy(sc_hbm.at[pl.ds(1 * NPAD, NPAD)], p1_t)
        pltpu.sync_copy(sc_hbm.at[pl.ds(2 * NPAD, NPAD)], q0_t)
        pltpu.sync_copy(sc_hbm.at[pl.ds(3 * NPAD, NPAD)], q1_t)

        def zrs(i, carry):
            rs0[pl.ds(i * 16, 16)] = jnp.zeros((16,), jnp.float32)
            rs1[pl.ds(i * 16, 16)] = jnp.zeros((16,), jnp.float32)
            return carry
        lax.fori_loop(0, NPAD // 16, zrs, 0, unroll=4)

        def chunk(j, carry):
            src_row = src_v.at[j]
            rel_row = rel_v.at[j]
            dst_row = dst_v.at[j]
            for i in range(C // 16):
                sl = pl.ds(i * 16, 16)
                s16 = src_row[sl]
                r16 = rel_row[sl]
                d16 = dst_row[sl]
                for hd, (p_t, q_t, rs_t, ee_t) in enumerate(
                        [(p0_t, q0_t, rs0, ee0_v), (p1_t, q1_t, rs1, ee1_v)]):
                    pv = plsc.load_gather(p_t, [s16])
                    qv = plsc.load_gather(q_t, [r16])
                    ev = pv + qv
                    lr = jnp.where(ev >= 0.0, ev, ev * 0.2)
                    ee = jnp.exp(-lr)
                    ee_t[sl] = ee
                    plsc.addupdate_scatter(rs_t, [d16], ee)
            gchunk = wid * CH_PER_TILE + j
            pltpu.sync_copy(ee0_v, ee_out.at[pl.ds(gchunk * C, C)])
            pltpu.sync_copy(ee1_v, ee_out.at[pl.ds((NCHUNK + gchunk) * C, C)])
            return carry
        lax.fori_loop(0, CH_PER_TILE, chunk, 0)

        pltpu.sync_copy(rs0, rs_out.at[pl.ds((0 * 32 + wid) * NPAD, NPAD)])
        pltpu.sync_copy(rs1, rs_out.at[pl.ds((1 * 32 + wid) * NPAD, NPAD)])

    return k(scores_flat, src2d, dst2d, rel2d)


def _sc_msg(h_pad, ir2, ee_flat, w_vec, src2d, dst2d, rel2d, zeros_rows):
    @functools.partial(
        pl.kernel,
        out_type=jax.ShapeDtypeStruct((2, 2, NPAD, F), jnp.float32),
        mesh=_mesh(),
        scratch_types=[
            pltpu.VMEM((CH_PER_TILE, C), jnp.int32),
            pltpu.VMEM((CH_PER_TILE, C), jnp.int32),
            pltpu.VMEM((CH_PER_TILE, C), jnp.int32),
            pltpu.VMEM((C, F), jnp.float32),
            pltpu.VMEM((C, F), jnp.float32),
            pltpu.VMEM((C,), jnp.float32),
            pltpu.VMEM((F,), jnp.float32),
            pltpu.VMEM_SHARED((NPAD, F), jnp.float32),
            pltpu.SemaphoreType.DMA,
            pltpu.SemaphoreType.DMA,
            pltpu.SemaphoreType.DMA,
        ],
        compiler_params=pltpu.CompilerParams(needs_layout_passes=False),
    )
    def k(h_hbm, ir2_hbm, ee_hbm, w_hbm, src_hbm, dst_hbm, rel_hbm, z_hbm,
          acc_out,
          src_v, dst_v, rel_v, hs_v, ir_v, ee_v, w_v, acc_sh,
          sem1, sem2, sem3):
        cid = lax.axis_index("c")
        sid = lax.axis_index("s")
        wid = sid * 2 + cid
        row0 = sid * ROWS_PER_TILE
        pltpu.sync_copy(src_hbm.at[pl.ds(wid * CH_PER_TILE, CH_PER_TILE), :], src_v)
        pltpu.sync_copy(dst_hbm.at[pl.ds(wid * CH_PER_TILE, CH_PER_TILE), :], dst_v)
        pltpu.sync_copy(rel_hbm.at[pl.ds(wid * CH_PER_TILE, CH_PER_TILE), :], rel_v)
        pltpu.sync_copy(w_hbm, w_v)

        for hd in range(2):
            pltpu.sync_copy(z_hbm.at[pl.ds(row0, ROWS_PER_TILE), :],
                            acc_sh.at[pl.ds(row0, ROWS_PER_TILE), :])
            plsc.subcore_barrier()

            def chunk(j, carry):
                g1 = pltpu.async_copy(h_hbm.at[src_v.at[j]], hs_v, sem1)
                g2 = pltpu.async_copy(ir2_hbm.at[rel_v.at[j]], ir_v, sem2)
                gchunk = (hd * NCHUNK) + wid * CH_PER_TILE + j
                g3 = pltpu.async_copy(ee_hbm.at[pl.ds(gchunk * C, C)], ee_v, sem3)
                g1.wait()
                g2.wait()
                g3.wait()

                def edge(e, c2):
                    eidx = jax.lax.broadcast(e, (16,))
                    s = plsc.load_gather(ee_v, [eidx])
                    hs_row = hs_v.at[e]
                    ir_row = ir_v.at[e]
                    for kk in range(F // 16):
                        sl = pl.ds(kk * 16, 16)
                        hv = hs_row[sl]
                        if hd == 1:
                            hv = hv * w_v[sl]
                        hs_row[sl] = (hv - ir_row[sl]) * s
                    return c2
                lax.fori_loop(0, C, edge, 0, unroll=2)
                pltpu.sync_copy(hs_v, acc_sh.at[dst_v.at[j]], add=True)
                return carry
            lax.fori_loop(0, CH_PER_TILE, chunk, 0)

            plsc.subcore_barrier()
            pltpu.sync_copy(acc_sh.at[pl.ds(row0, ROWS_PER_TILE), :],
                            acc_out.at[hd, cid, pl.ds(row0, ROWS_PER_TILE), :])

    return k(h_pad, ir2, ee_flat, w_vec, src2d, dst2d, rel2d, zeros_rows)


def _tc_d(acc, rsum_t):
    def body(acc_ref, rs_ref, out_ref):
        rs_sum = jnp.sum(rs_ref[...], axis=2)
        out_ref[0] = (acc_ref[0, 0] + acc_ref[0, 1]) / rs_sum[:, 0:1]
        out_ref[1] = (acc_ref[1, 0] + acc_ref[1, 1]) / rs_sum[:, 1:2]

    return pl.pallas_call(
        body,
        grid=(NB,),
        in_specs=[
            pl.BlockSpec((2, 2, BR, F), lambda i: (0, 0, i, 0)),
            pl.BlockSpec((BR, 2, 32), lambda i: (i, 0, 0)),
        ],
        out_specs=pl.BlockSpec((2, BR, F), lambda i: (0, i, 0)),
        out_shape=jax.ShapeDtypeStruct((2, NPAD, F), jnp.float32),
    )(acc, rsum_t)


def kernel(h, A, r_count, w, a_src_dst):
    dst = A[0]
    rel = A[1]
    src = A[2]
    npad_e = EPAD - E
    pad0 = jnp.zeros((npad_e,), jnp.int32)
    padN = jnp.full((npad_e,), N, jnp.int32)
    src2d = jnp.concatenate([src, pad0]).reshape(NCHUNK, C)
    dst2d = jnp.concatenate([dst, padN]).reshape(NCHUNK, C)
    dstg2d = jnp.concatenate([dst, pad0]).reshape(NCHUNK, C)
    rel2d = jnp.concatenate([rel, padN]).reshape(NCHUNK, C)

    h_pad = jnp.concatenate([h, jnp.zeros((NPAD - N, F), h.dtype)], axis=0)
    rc_pad = jnp.concatenate([r_count, jnp.ones((NPAD - N,), r_count.dtype)])
    rc_pad = rc_pad.reshape(NPAD, 1)
    zeros_rows = jnp.zeros((NPAD, F), jnp.float32)

    a00 = a_src_dst[0, 0, :, 0]
    a10 = a_src_dst[1, 0, :, 0]
    a01 = a_src_dst[0, 1, :, 0]
    a11 = a_src_dst[1, 1, :, 0]
    w0 = w[0, 0]
    a_flat = jnp.stack(
        [a00, a10, a01, a11, w0,
         jnp.zeros((F,), jnp.float32), jnp.zeros((F,), jnp.float32),
         jnp.zeros((F,), jnp.float32)], axis=0)

    part = _sc_pass1(h_pad, src2d, dstg2d, rel2d, zeros_rows)
    inputr, edge_r, csum = _tc_a(part, rc_pad)
    g = _tc_b(inputr, edge_r, csum)
    inputr2, scores = _tc_c(edge_r, g, h_pad, a_flat)
    scores_flat = scores.T[:4].reshape(-1)
    ee_flat, rsum = _sc_scores(scores_flat, src2d, dst2d, rel2d)
    acc = _sc_msg(h_pad, inputr2, ee_flat, w0, src2d, dst2d, rel2d, zeros_rows)
    rsum_t = jnp.transpose(rsum.reshape(2, 32, NPAD), (2, 0, 1))
    out_pad = _tc_d(acc, rsum_t)
    return out_pad[:, :N, :]


# speedup vs baseline: 2.7358x; 1.2686x over previous
import functools

import jax
import jax.numpy as jnp
from jax import lax
from jax.experimental import pallas as pl
from jax.experimental.pallas import tpu as pltpu
from jax.experimental.pallas import tpu_sc as plsc

N = 10000
F = 128
E = 160000
NPAD = 10112
EPAD = 163840
C = 64
NCHUNK = EPAD // C
CH_PER_TILE = NCHUNK // 32
ROWS_PER_TILE = NPAD // 16
NB = 4
BR = NPAD // NB

_mesh = functools.partial(
    plsc.VectorSubcoreMesh, core_axis_name="c", subcore_axis_name="s",
    num_cores=2, num_subcores=16)


def _tc_neg(h_pad):
    def body(h_ref, o_ref):
        o_ref[...] = -h_ref[...]

    return pl.pallas_call(
        body,
        grid=(NB,),
        in_specs=[pl.BlockSpec((BR, F), lambda i: (i, 0))],
        out_specs=pl.BlockSpec((BR, F), lambda i: (i, 0)),
        out_shape=jax.ShapeDtypeStruct((NPAD, F), jnp.float32),
    )(h_pad)


def _sc_pass1(h_pad, h_neg, src1d, dst1d, rel2d, zeros_rows):
    @functools.partial(
        pl.kernel,
        out_type=jax.ShapeDtypeStruct((2, NPAD, F), jnp.float32),
        mesh=_mesh(),
        scratch_types=[
            pltpu.VMEM((EPAD // 32,), jnp.int32),
            pltpu.VMEM((EPAD // 32,), jnp.int32),
            pltpu.VMEM((CH_PER_TILE, C), jnp.int32),
            pltpu.VMEM((C, F), jnp.float32),
            pltpu.VMEM((C, F), jnp.float32),
            pltpu.VMEM_SHARED((NPAD, F), jnp.float32),
            pltpu.SemaphoreType.DMA,
            pltpu.SemaphoreType.DMA,
        ],
    )
    def k(h_hbm, hn_hbm, src_hbm, dst_hbm, rel_hbm, z_hbm, out_hbm,
          src_v, dst_v, rel_v, r0_v, r1_v, acc_sh, sem0, sem1):
        cid = lax.axis_index("c")
        sid = lax.axis_index("s")
        wid = sid * 2 + cid
        row0 = sid * ROWS_PER_TILE
        epw = EPAD // 32
        pltpu.sync_copy(src_hbm.at[pl.ds(wid * epw, epw)], src_v)
        pltpu.sync_copy(dst_hbm.at[pl.ds(wid * epw, epw)], dst_v)
        pltpu.sync_copy(rel_hbm.at[pl.ds(wid * CH_PER_TILE, CH_PER_TILE), :], rel_v)
        pltpu.sync_copy(z_hbm.at[pl.ds(row0, ROWS_PER_TILE), :],
                        acc_sh.at[pl.ds(row0, ROWS_PER_TILE), :])
        plsc.subcore_barrier()

        bufs = ((r0_v, sem0), (r1_v, sem1))

        for tab_hbm, idx_v in ((h_hbm, src_v), (hn_hbm, dst_v)):

            def start(j, s):
                r_v, sem = bufs[s]
                pltpu.async_copy(tab_hbm.at[idx_v.at[pl.ds(j * C, C)]], r_v, sem)

            def finish(j, s):
                r_v, sem = bufs[s]
                pltpu.make_async_copy(tab_hbm.at[idx_v.at[pl.ds(j * C, C)]],
                                      r_v, sem).wait()
                pltpu.sync_copy(r_v, acc_sh.at[rel_v.at[j]], add=True)

            start(0, 0)

            def pair(t, carry):
                j0 = 2 * t
                start(j0 + 1, 1)
                finish(j0, 0)

                @pl.when(t < CH_PER_TILE // 2 - 1)
                def _():
                    start(j0 + 2, 0)
                finish(j0 + 1, 1)
                return carry
            lax.fori_loop(0, CH_PER_TILE // 2, pair, 0)

        plsc.subcore_barrier()
        pltpu.sync_copy(acc_sh.at[pl.ds(row0, ROWS_PER_TILE), :],
                        out_hbm.at[cid, pl.ds(row0, ROWS_PER_TILE), :])

    return k(h_pad, h_neg, src1d, dst1d, rel2d, zeros_rows)


def _tc_a(part, rc_pad):
    def body(part_ref, rc_ref, ir_ref, er_ref, c_ref):
        raw = part_ref[0] + part_ref[1]
        inv = 1.0 / rc_ref[...]
        ir = raw * inv
        er = ir * inv
        ir_ref[...] = ir
        er_ref[...] = er

        @pl.when(pl.program_id(0) == 0)
        def _():
            c_ref[...] = jnp.zeros_like(c_ref)
        c_ref[0:1, :] += jnp.sum(er, axis=0, keepdims=True)

    return pl.pallas_call(
        body,
        grid=(NB,),
        in_specs=[
            pl.BlockSpec((2, BR, F), lambda i: (0, i, 0)),
            pl.BlockSpec((BR, 1), lambda i: (i, 0)),
        ],
        out_specs=[
            pl.BlockSpec((BR, F), lambda i: (i, 0)),
            pl.BlockSpec((BR, F), lambda i: (i, 0)),
            pl.BlockSpec((8, F), lambda i: (0, 0)),
        ],
        out_shape=[
            jax.ShapeDtypeStruct((NPAD, F), jnp.float32),
            jax.ShapeDtypeStruct((NPAD, F), jnp.float32),
            jax.ShapeDtypeStruct((8, F), jnp.float32),
        ],
    )(part, rc_pad)


def _tc_b(inputr, edge_r, csum):
    def body(ir_ref, er_ref, c_ref, g_ref):
        er = er_ref[...]
        cvec = c_ref[0:1, :]
        rowsum = jax.lax.dot_general(
            er, cvec, (((1,), (1,)), ((), ())),
            preferred_element_type=jnp.float32, precision=jax.lax.Precision.HIGHEST)
        u = jnp.where(rowsum != 0.0, ir_ref[...] / rowsum, 0.0)

        @pl.when(pl.program_id(0) == 0)
        def _():
            g_ref[...] = jnp.zeros_like(g_ref)
        g_ref[...] += jax.lax.dot_general(
            er, u, (((0,), (0,)), ((), ())),
            preferred_element_type=jnp.float32, precision=jax.lax.Precision.HIGHEST)

    return pl.pallas_call(
        body,
        grid=(NB,),
        in_specs=[
            pl.BlockSpec((BR, F), lambda i: (i, 0)),
            pl.BlockSpec((BR, F), lambda i: (i, 0)),
            pl.BlockSpec((8, F), lambda i: (0, 0)),
        ],
        out_specs=pl.BlockSpec((F, F), lambda i: (0, 0)),
        out_shape=jax.ShapeDtypeStruct((F, F), jnp.float32),
    )(inputr, edge_r, csum)


def _tc_c(edge_r, g, h_pad, a_flat):
    def body(er_ref, g_ref, h_ref, af_ref, ir2n_ref, hw_ref, sc_ref):
        er = er_ref[...]
        ir2 = jnp.dot(er, g_ref[...], preferred_element_type=jnp.float32, precision=jax.lax.Precision.HIGHEST)
        hblk = h_ref[...]
        af = af_ref[...]
        w0 = af[4:5, :]
        hw = hblk * w0
        ir2n_ref[...] = -ir2
        hw_ref[...] = hw
        p0 = jax.lax.dot_general(hblk, af[0:1, :], (((1,), (1,)), ((), ())),
                                 preferred_element_type=jnp.float32, precision=jax.lax.Precision.HIGHEST)
        p1 = jax.lax.dot_general(hw, af[1:2, :], (((1,), (1,)), ((), ())),
                                 preferred_element_type=jnp.float32, precision=jax.lax.Precision.HIGHEST)
        q0 = jax.lax.dot_general(ir2, af[2:3, :], (((1,), (1,)), ((), ())),
                                 preferred_element_type=jnp.float32, precision=jax.lax.Precision.HIGHEST)
        q1 = jax.lax.dot_general(ir2, af[3:4, :], (((1,), (1,)), ((), ())),
                                 preferred_element_type=jnp.float32, precision=jax.lax.Precision.HIGHEST)
        z = jnp.zeros_like(p0)
        sc_ref[...] = jnp.concatenate([p0, p1, q0, q1, z, z, z, z], axis=1)

    return pl.pallas_call(
        body,
        grid=(NB,),
        in_specs=[
            pl.BlockSpec((BR, F), lambda i: (i, 0)),
            pl.BlockSpec((F, F), lambda i: (0, 0)),
            pl.BlockSpec((BR, F), lambda i: (i, 0)),
            pl.BlockSpec((8, F), lambda i: (0, 0)),
        ],
        out_specs=[
            pl.BlockSpec((BR, F), lambda i: (i, 0)),
            pl.BlockSpec((BR, F), lambda i: (i, 0)),
            pl.BlockSpec((BR, 8), lambda i: (i, 0)),
        ],
        out_shape=[
            jax.ShapeDtypeStruct((NPAD, F), jnp.float32),
            jax.ShapeDtypeStruct((NPAD, F), jnp.float32),
            jax.ShapeDtypeStruct((NPAD, 8), jnp.float32),
        ],
    )(edge_r, g, h_pad, a_flat)


def _sc_scores(scores_flat, src2d, dst2d, rel2d):
    @functools.partial(
        pl.kernel,
        out_type=(
            jax.ShapeDtypeStruct((2 * EPAD,), jnp.float32),
            jax.ShapeDtypeStruct((2 * 32 * NPAD,), jnp.float32),
        ),
        mesh=_mesh(),
        scratch_types=[
            pltpu.VMEM((CH_PER_TILE, C), jnp.int32),
            pltpu.VMEM((CH_PER_TILE, C), jnp.int32),
            pltpu.VMEM((CH_PER_TILE, C), jnp.int32),
            pltpu.VMEM((NPAD,), jnp.float32),
            pltpu.VMEM((NPAD,), jnp.float32),
            pltpu.VMEM((NPAD,), jnp.float32),
            pltpu.VMEM((NPAD,), jnp.float32),
            pltpu.VMEM((NPAD,), jnp.float32),
            pltpu.VMEM((NPAD,), jnp.float32),
            pltpu.VMEM((C,), jnp.float32),
            pltpu.VMEM((C,), jnp.float32),
        ],
        compiler_params=pltpu.CompilerParams(needs_layout_passes=False),
    )
    def k(sc_hbm, src_hbm, dst_hbm, rel_hbm, ee_out, rs_out,
          src_v, dst_v, rel_v, p0_t, q0_t, p1_t, q1_t, rs0, rs1, ee0_v, ee1_v):
        cid = lax.axis_index("c")
        sid = lax.axis_index("s")
        wid = sid * 2 + cid
        pltpu.sync_copy(src_hbm.at[pl.ds(wid * CH_PER_TILE, CH_PER_TILE), :], src_v)
        pltpu.sync_copy(dst_hbm.at[pl.ds(wid * CH_PER_TILE, CH_PER_TILE), :], dst_v)
        pltpu.sync_copy(rel_hbm.at[pl.ds(wid * CH_PER_TILE, CH_PER_TILE), :], rel_v)
        pltpu.sync_copy(sc_hbm.at[pl.ds(0 * NPAD, NPAD)], p0_t)
        pltpu.sync_copy(sc_hbm.at[pl.ds(1 * NPAD, NPAD)], p1_t)
        pltpu.sync_copy(sc_hbm.at[pl.ds(2 * NPAD, NPAD)], q0_t)
        pltpu.sync_copy(sc_hbm.at[pl.ds(3 * NPAD, NPAD)], q1_t)

        def zrs(i, carry):
            rs0[pl.ds(i * 16, 16)] = jnp.zeros((16,), jnp.float32)
            rs1[pl.ds(i * 16, 16)] = jnp.zeros((16,), jnp.float32)
            return carry
        lax.fori_loop(0, NPAD // 16, zrs, 0, unroll=4)

        def chunk(j, carry):
            src_row = src_v.at[j]
            rel_row = rel_v.at[j]
            dst_row = dst_v.at[j]
            for i in range(C // 16):
                sl = pl.ds(i * 16, 16)
                s16 = src_row[sl]
                r16 = rel_row[sl]
                d16 = dst_row[sl]
                for hd, (p_t, q_t, rs_t, ee_t) in enumerate(
                        [(p0_t, q0_t, rs0, ee0_v), (p1_t, q1_t, rs1, ee1_v)]):
                    pv = plsc.load_gather(p_t, [s16])
                    qv = plsc.load_gather(q_t, [r16])
                    ev = pv + qv
                    lr = jnp.where(ev >= 0.0, ev, ev * 0.2)
                    ee = jnp.exp(-lr)
                    ee_t[sl] = ee
                    plsc.addupdate_scatter(rs_t, [d16], ee)
            gchunk = wid * CH_PER_TILE + j
            pltpu.sync_copy(ee0_v, ee_out.at[pl.ds(gchunk * C, C)])
            pltpu.sync_copy(ee1_v, ee_out.at[pl.ds((NCHUNK + gchunk) * C, C)])
            return carry
        lax.fori_loop(0, CH_PER_TILE, chunk, 0)

        pltpu.sync_copy(rs0, rs_out.at[pl.ds((0 * 32 + wid) * NPAD, NPAD)])
        pltpu.sync_copy(rs1, rs_out.at[pl.ds((1 * 32 + wid) * NPAD, NPAD)])

    return k(scores_flat, src2d, dst2d, rel2d)


def _sc_msg(h_pad, hw, ir2n, ee_flat, src1d, dst2d, rel1d, zeros_rows):
    @functools.partial(
        pl.kernel,
        out_type=jax.ShapeDtypeStruct((2, 2, NPAD, F), jnp.float32),
        mesh=_mesh(),
        scratch_types=[
            pltpu.VMEM((EPAD // 32,), jnp.int32),
            pltpu.VMEM((EPAD // 32,), jnp.int32),
            pltpu.VMEM((CH_PER_TILE, C), jnp.int32),
            pltpu.VMEM((C, F), jnp.float32),
            pltpu.VMEM((C, F), jnp.float32),
            pltpu.VMEM((C,), jnp.float32),
            pltpu.VMEM((C,), jnp.float32),
            pltpu.VMEM_SHARED((NPAD, F), jnp.float32),
            pltpu.SemaphoreType.DMA,
            pltpu.SemaphoreType.DMA,
        ],
        compiler_params=pltpu.CompilerParams(needs_layout_passes=False),
    )
    def k(h_hbm, hw_hbm, ir2n_hbm, ee_hbm, src_hbm, dst_hbm, rel_hbm, z_hbm,
          acc_out,
          src_v, rel_v, dst_v, r0_v, r1_v, ee0_v, ee1_v,
          acc_sh, sem0, sem1):
        cid = lax.axis_index("c")
        sid = lax.axis_index("s")
        wid = sid * 2 + cid
        row0 = sid * ROWS_PER_TILE
        epw = EPAD // 32
        pltpu.sync_copy(src_hbm.at[pl.ds(wid * epw, epw)], src_v)
        pltpu.sync_copy(rel_hbm.at[pl.ds(wid * epw, epw)], rel_v)
        pltpu.sync_copy(dst_hbm.at[pl.ds(wid * CH_PER_TILE, CH_PER_TILE), :], dst_v)

        bufs = ((r0_v, ee0_v, sem0), (r1_v, ee1_v, sem1))

        for hd in range(2):
            pltpu.sync_copy(z_hbm.at[pl.ds(row0, ROWS_PER_TILE), :],
                            acc_sh.at[pl.ds(row0, ROWS_PER_TILE), :])
            plsc.subcore_barrier()

            for tab_hbm, idx_v in (((h_hbm if hd == 0 else hw_hbm), src_v),
                                   (ir2n_hbm, rel_v)):

                def start(j, s):
                    r_v, ee_v, sem = bufs[s]
                    gchunk = (hd * NCHUNK) + wid * CH_PER_TILE + j
                    pltpu.async_copy(tab_hbm.at[idx_v.at[pl.ds(j * C, C)]], r_v, sem)
                    pltpu.async_copy(ee_hbm.at[pl.ds(gchunk * C, C)], ee_v, sem)

                def finish(j, s):
                    r_v, ee_v, sem = bufs[s]
                    gchunk = (hd * NCHUNK) + wid * CH_PER_TILE + j
                    pltpu.make_async_copy(tab_hbm.at[idx_v.at[pl.ds(j * C, C)]],
                                          r_v, sem).wait()
                    pltpu.make_async_copy(ee_hbm.at[pl.ds(gchunk * C, C)], ee_v,
                                          sem).wait()

                    def edge(e, c2):
                        eidx = jax.lax.broadcast(e, (16,))
                        s_ = plsc.load_gather(ee_v, [eidx])
                        r_row = r_v.at[e]
                        for kk in range(F // 16):
                            sl = pl.ds(kk * 16, 16)
                            r_row[sl] = r_row[sl] * s_
                        return c2
                    lax.fori_loop(0, C, edge, 0, unroll=4)
                    pltpu.sync_copy(r_v, acc_sh.at[dst_v.at[j]], add=True)

                start(0, 0)

                def pair(t, carry):
                    j0 = 2 * t
                    start(j0 + 1, 1)
                    finish(j0, 0)

                    @pl.when(t < CH_PER_TILE // 2 - 1)
                    def _():
                        start(j0 + 2, 0)
                    finish(j0 + 1, 1)
                    return carry
                lax.fori_loop(0, CH_PER_TILE // 2, pair, 0)

            plsc.subcore_barrier()
            pltpu.sync_copy(acc_sh.at[pl.ds(row0, ROWS_PER_TILE), :],
                            acc_out.at[hd, cid, pl.ds(row0, ROWS_PER_TILE), :])

    return k(h_pad, hw, ir2n, ee_flat, src1d, dst2d, rel1d, zeros_rows)


def _tc_d(acc, rsum_t):
    def body(acc_ref, rs_ref, out_ref):
        rs_sum = jnp.sum(rs_ref[...], axis=2)
        out_ref[0] = (acc_ref[0, 0] + acc_ref[0, 1]) / rs_sum[:, 0:1]
        out_ref[1] = (acc_ref[1, 0] + acc_ref[1, 1]) / rs_sum[:, 1:2]

    return pl.pallas_call(
        body,
        grid=(NB,),
        in_specs=[
            pl.BlockSpec((2, 2, BR, F), lambda i: (0, 0, i, 0)),
            pl.BlockSpec((BR, 2, 32), lambda i: (i, 0, 0)),
        ],
        out_specs=pl.BlockSpec((2, BR, F), lambda i: (0, i, 0)),
        out_shape=jax.ShapeDtypeStruct((2, NPAD, F), jnp.float32),
    )(acc, rsum_t)


def kernel(h, A, r_count, w, a_src_dst):
    dst = A[0]
    rel = A[1]
    src = A[2]
    npad_e = EPAD - E
    pad0 = jnp.zeros((npad_e,), jnp.int32)
    padN = jnp.full((npad_e,), N, jnp.int32)
    src1d = jnp.concatenate([src, pad0])
    dst1d = jnp.concatenate([dst, pad0])
    rel1d = jnp.concatenate([rel, padN])
    src2d = src1d.reshape(NCHUNK, C)
    dst2d = jnp.concatenate([dst, padN]).reshape(NCHUNK, C)
    rel2d = rel1d.reshape(NCHUNK, C)

    h_pad = jnp.concatenate([h, jnp.zeros((NPAD - N, F), h.dtype)], axis=0)
    rc_pad = jnp.concatenate([r_count, jnp.ones((NPAD - N,), r_count.dtype)])
    rc_pad = rc_pad.reshape(NPAD, 1)
    zeros_rows = jnp.zeros((NPAD, F), jnp.float32)

    a00 = a_src_dst[0, 0, :, 0]
    a10 = a_src_dst[1, 0, :, 0]
    a01 = a_src_dst[0, 1, :, 0]
    a11 = a_src_dst[1, 1, :, 0]
    w0 = w[0, 0]
    a_flat = jnp.stack(
        [a00, a10, a01, a11, w0,
         jnp.zeros((F,), jnp.float32), jnp.zeros((F,), jnp.float32),
         jnp.zeros((F,), jnp.float32)], axis=0)

    h_neg = _tc_neg(h_pad)
    part = _sc_pass1(h_pad, h_neg, src1d, dst1d, rel2d, zeros_rows)
    inputr, edge_r, csum = _tc_a(part, rc_pad)
    g = _tc_b(inputr, edge_r, csum)
    ir2n, hw, scores = _tc_c(edge_r, g, h_pad, a_flat)
    scores_flat = scores.T[:4].reshape(-1)
    ee_flat, rsum = _sc_scores(scores_flat, src2d, dst2d, rel2d)
    acc = _sc_msg(h_pad, hw, ir2n, ee_flat, src1d, dst2d, rel1d, zeros_rows)
    rsum_t = jnp.transpose(rsum.reshape(2, 32, NPAD), (2, 0, 1))
    out_pad = _tc_d(acc, rsum_t)
    return out_pad[:, :N, :]
